# baseline (speedup 1.0000x reference)
import numpy as np
import jax
import jax.numpy as jnp
from jax import lax
from jax.experimental import pallas as pl
from jax.experimental.pallas import tpu as pltpu

_VOX = np.array([0.16, 0.16, 4.0], dtype=np.float32)
_PCR = np.array([0.0, -39.68, -3.0, 69.12, 39.68, 1.0], dtype=np.float32)
_PPV = 35
_V = 20000
_SENT = np.int32(np.iinfo(np.int32).max)
_GRID = ((_PCR[3:] - _PCR[:3]) / _VOX).astype(np.int32)
_R, _C = 750, 160


def _ew_body(sig_ref, x_ref, y_ref, z_ref, vx_ref, vy_ref, vz_ref,
             d_ref, w_ref):
    sig = jnp.maximum(sig_ref[0, 0], np.float32(0.001))
    x = x_ref[...]
    y = y_ref[...]
    z = z_ref[...]
    cx = (vx_ref[...].astype(jnp.float32) + 0.5) * _VOX[0] + _PCR[0]
    cy = (vy_ref[...].astype(jnp.float32) + 0.5) * _VOX[1] + _PCR[1]
    cz = (vz_ref[...].astype(jnp.float32) + 0.5) * _VOX[2] + _PCR[2]
    dx = x - cx
    dy = y - cy
    dz = z - cz
    d = jnp.sqrt(dx * dx + dy * dy + dz * dz)
    d_ref[...] = d
    w_ref[...] = jnp.exp(-0.5 * (d / sig) ** 2)


def _dist_weight(points, vi, sigma):
    x2 = points[:, 0].reshape(_R, _C)
    y2 = points[:, 1].reshape(_R, _C)
    z2 = points[:, 2].reshape(_R, _C)
    vx2 = vi[:, 0].reshape(_R, _C)
    vy2 = vi[:, 1].reshape(_R, _C)
    vz2 = vi[:, 2].reshape(_R, _C)
    sig2 = jnp.reshape(sigma, (1, 1))
    vspec = pl.BlockSpec((_R, _C), lambda: (0, 0))
    d2, w2 = pl.pallas_call(
        _ew_body,
        out_shape=[jax.ShapeDtypeStruct((_R, _C), jnp.float32)] * 2,
        in_specs=[pl.BlockSpec(memory_space=pltpu.SMEM)] + [vspec] * 6,
        out_specs=[vspec] * 2,
    )(sig2, x2, y2, z2, vx2, vy2, vz2)
    return d2.reshape(-1), w2.reshape(-1)


def kernel(points, sigma):
    n = points.shape[0]
    vs = jnp.asarray(_VOX, dtype=points.dtype)
    pcr = jnp.asarray(_PCR, dtype=points.dtype)
    vi = jnp.floor((points[:, :3] - pcr[:3]) / vs).astype(jnp.int32)
    grid = jnp.asarray(_GRID)
    valid = jnp.all((vi >= 0) & (vi < grid), axis=1)
    hsh = vi[:, 0] * 1000000 + vi[:, 1] * 1000 + vi[:, 2]
    hsh = jnp.where(valid, hsh, _SENT)

    dist, w = _dist_weight(points, vi, sigma)
    dbits = lax.bitcast_convert_type(dist, jnp.int32)

    pad = 131072
    hp = jnp.concatenate([hsh, jnp.full((pad - n,), _SENT, jnp.int32)])
    dp = jnp.concatenate(
        [dbits, jnp.full((pad - n,), np.int32(0x7FFFFFFF), jnp.int32)])
    ord1 = jnp.argsort(dp, stable=True)
    ord2 = jnp.argsort(hp[ord1], stable=True)
    order = ord1[ord2][:n]
    h_s = hsh[order]
    w_s = w[order]
    pts_s = points[order]

    pos = jnp.arange(n, dtype=jnp.int32)
    newseg = jnp.concatenate(
        [jnp.ones((1,), jnp.bool_), h_s[1:] != h_s[:-1]])
    segid = jnp.cumsum(newseg.astype(jnp.int32)) - 1
    start = lax.cummax(jnp.where(newseg, pos, 0), axis=0)
    is_last = jnp.concatenate(
        [newseg[1:], jnp.ones((1,), jnp.bool_)])
    end = lax.cummin(jnp.where(is_last, pos, n - 1), axis=0, reverse=True)
    rank = pos - start

    validp = h_s != _SENT
    segc = jnp.minimum(segid, _V)
    kept = validp & (segid < _V) & (rank < _PPV)
    wk = jnp.where(kept, w_s, jnp.zeros_like(w_s))
    wsumv = jax.ops.segment_sum(wk, segc, num_segments=_V + 1,
                                indices_are_sorted=True)
    wn = w_s / (wsumv[segc] + 1e-6)

    flat = jnp.where(kept, segid * _PPV + rank, _V * _PPV + pos)
    feat = jnp.zeros((_V * _PPV, points.shape[1]), points.dtype).at[flat].set(
        pts_s * wn[:, None], mode="drop", unique_indices=True)
    feat = feat.reshape(_V, _PPV, points.shape[1])

    first_ok = newseg & validp & (segid < _V)
    tgt = jnp.where(first_ok, segid, _V + pos)
    cxi = h_s // 1000000
    cyi = (h_s % 1000000) // 1000
    czi = h_s % 1000
    coords_init = jnp.broadcast_to(
        jnp.array([2147, 483, 647], jnp.int32), (_V, 3))
    coords = coords_init.at[tgt].set(
        jnp.stack([cxi, cyi, czi], axis=1), mode="drop", unique_indices=True)

    seg_len = end - start + 1
    nppv = jnp.zeros((_V,), jnp.int32).at[tgt].set(
        jnp.minimum(seg_len, _PPV), mode="drop", unique_indices=True)
    return feat, coords, nppv


# speedup vs baseline: 1.0081x; 1.0081x over previous
import numpy as np
import jax
import jax.numpy as jnp
from jax import lax
from jax.experimental import pallas as pl
from jax.experimental.pallas import tpu as pltpu

_VOX = np.array([0.16, 0.16, 4.0], dtype=np.float32)
_PCR = np.array([0.0, -39.68, -3.0, 69.12, 39.68, 1.0], dtype=np.float32)
_PPV = 35
_V = 20000
_SENT = np.int32(np.iinfo(np.int32).max)
_GRID = ((_PCR[3:] - _PCR[:3]) / _VOX).astype(np.int32)
_R, _C = 750, 160


def _ew_body(sig_ref, x_ref, y_ref, z_ref, vx_ref, vy_ref, vz_ref,
             d_ref, w_ref):
    sig = jnp.maximum(sig_ref[0, 0], np.float32(0.001))
    x = x_ref[...]
    y = y_ref[...]
    z = z_ref[...]
    cx = (vx_ref[...].astype(jnp.float32) + 0.5) * _VOX[0] + _PCR[0]
    cy = (vy_ref[...].astype(jnp.float32) + 0.5) * _VOX[1] + _PCR[1]
    cz = (vz_ref[...].astype(jnp.float32) + 0.5) * _VOX[2] + _PCR[2]
    dx = x - cx
    dy = y - cy
    dz = z - cz
    d = jnp.sqrt(dx * dx + dy * dy + dz * dz)
    d_ref[...] = d
    w_ref[...] = jnp.exp(-0.5 * (d / sig) ** 2)


def _dist_weight(points, vi, sigma):
    x2 = points[:, 0].reshape(_R, _C)
    y2 = points[:, 1].reshape(_R, _C)
    z2 = points[:, 2].reshape(_R, _C)
    vx2 = vi[:, 0].reshape(_R, _C)
    vy2 = vi[:, 1].reshape(_R, _C)
    vz2 = vi[:, 2].reshape(_R, _C)
    sig2 = jnp.reshape(sigma, (1, 1))
    vspec = pl.BlockSpec((_R, _C), lambda: (0, 0))
    d2, w2 = pl.pallas_call(
        _ew_body,
        out_shape=[jax.ShapeDtypeStruct((_R, _C), jnp.float32)] * 2,
        in_specs=[pl.BlockSpec(memory_space=pltpu.SMEM)] + [vspec] * 6,
        out_specs=[vspec] * 2,
    )(sig2, x2, y2, z2, vx2, vy2, vz2)
    return d2.reshape(-1), w2.reshape(-1)


def kernel(points, sigma):
    n = points.shape[0]
    vs = jnp.asarray(_VOX, dtype=points.dtype)
    pcr = jnp.asarray(_PCR, dtype=points.dtype)
    vi = jnp.floor((points[:, :3] - pcr[:3]) / vs).astype(jnp.int32)
    grid = jnp.asarray(_GRID)
    valid = jnp.all((vi >= 0) & (vi < grid), axis=1)
    hsh = vi[:, 0] * 1000000 + vi[:, 1] * 1000 + vi[:, 2]
    hsh = jnp.where(valid, hsh, _SENT)

    dist, w = _dist_weight(points, vi, sigma)
    dbits = lax.bitcast_convert_type(dist, jnp.int32)

    order = jnp.lexsort((dbits, hsh))
    h_s = hsh[order]
    w_s = w[order]
    pts_s = points[order]

    pos = jnp.arange(n, dtype=jnp.int32)
    newseg = jnp.concatenate(
        [jnp.ones((1,), jnp.bool_), h_s[1:] != h_s[:-1]])
    segid = jnp.cumsum(newseg.astype(jnp.int32)) - 1
    start = lax.cummax(jnp.where(newseg, pos, 0), axis=0)
    is_last = jnp.concatenate(
        [newseg[1:], jnp.ones((1,), jnp.bool_)])
    end = lax.cummin(jnp.where(is_last, pos, n - 1), axis=0, reverse=True)
    rank = pos - start

    validp = h_s != _SENT
    segc = jnp.minimum(segid, _V)
    kept = validp & (segid < _V) & (rank < _PPV)
    wk = jnp.where(kept, w_s, jnp.zeros_like(w_s))
    wsumv = jax.ops.segment_sum(wk, segc, num_segments=_V + 1,
                                indices_are_sorted=True)
    wn = w_s / (wsumv[segc] + 1e-6)

    flat = jnp.where(kept, segid * _PPV + rank, _V * _PPV + pos)
    feat = jnp.zeros((_V * _PPV, points.shape[1]), points.dtype).at[flat].set(
        pts_s * wn[:, None], mode="drop", unique_indices=True)
    feat = feat.reshape(_V, _PPV, points.shape[1])

    first_ok = newseg & validp & (segid < _V)
    tgt = jnp.where(first_ok, segid, _V + pos)
    cxi = h_s // 1000000
    cyi = (h_s % 1000000) // 1000
    czi = h_s % 1000
    coords_init = jnp.broadcast_to(
        jnp.array([2147, 483, 647], jnp.int32), (_V, 3))
    coords = coords_init.at[tgt].set(
        jnp.stack([cxi, cyi, czi], axis=1), mode="drop", unique_indices=True)

    seg_len = end - start + 1
    nppv = jnp.zeros((_V,), jnp.int32).at[tgt].set(
        jnp.minimum(seg_len, _PPV), mode="drop", unique_indices=True)
    return feat, coords, nppv
